# Initial kernel scaffold; baseline (speedup 1.0000x reference)
import jax
import jax.numpy as jnp
from jax.experimental import pallas as pl


def kernel(x, emb, W1, b1, W2, b2, W3, b3):
    raise NotImplementedError("write your pallas kernel here")


# speedup vs baseline: 1.0065x; 1.0065x over previous
import jax
import jax.numpy as jnp
from jax import lax
from jax.experimental import pallas as pl
from jax.experimental.pallas import tpu as pltpu
from jax.experimental.pallas import tpu_sc as plsc

B, L, D = 16384, 200, 64
LP = 208
NW = 32
RPW = B // NW
OUT_CHUNK = 16


def _sc_body(xp_hbm, emb_hbm, sum_hbm, idx_a, idx_b, rows_v, out_v, sem):
    ncores = lax.axis_size("c")
    wid = lax.axis_index("s") * ncores + lax.axis_index("c")
    base = wid * RPW

    def row_step(r, _):
        g = base + r
        pltpu.sync_copy(xp_hbm.at[g, pl.ds(0, 128)], idx_a)
        pltpu.sync_copy(xp_hbm.at[g, pl.ds(128, LP - 128)], idx_b)
        d1 = pltpu.async_copy(emb_hbm.at[idx_a], rows_v.at[pl.ds(0, 128)], sem)
        d2 = pltpu.async_copy(emb_hbm.at[idx_b], rows_v.at[pl.ds(128, LP - 128)], sem)
        d1.wait()
        d2.wait()

        def acc_step(j, accs):
            a0, a1, a2, a3 = accs
            return (a0 + rows_v[j, pl.ds(0, 16)],
                    a1 + rows_v[j, pl.ds(16, 16)],
                    a2 + rows_v[j, pl.ds(32, 16)],
                    a3 + rows_v[j, pl.ds(48, 16)])

        z = jnp.zeros((16,), jnp.float32)
        a0, a1, a2, a3 = lax.fori_loop(0, LP, acc_step, (z, z, z, z))

        s = r % OUT_CHUNK
        out_v[s, pl.ds(0, 16)] = a0
        out_v[s, pl.ds(16, 16)] = a1
        out_v[s, pl.ds(32, 16)] = a2
        out_v[s, pl.ds(48, 16)] = a3

        @pl.when(s == OUT_CHUNK - 1)
        def _flush():
            start = pl.multiple_of(g - (OUT_CHUNK - 1), OUT_CHUNK)
            pltpu.sync_copy(out_v, sum_hbm.at[pl.ds(start, OUT_CHUNK)])

        return 0

    lax.fori_loop(0, RPW, row_step, 0)


@jax.jit
def _sc_sum(xp, emb):
    mesh = plsc.VectorSubcoreMesh(core_axis_name="c", subcore_axis_name="s")
    f = pl.kernel(
        _sc_body,
        out_type=jax.ShapeDtypeStruct((B, D), jnp.float32),
        mesh=mesh,
        scratch_types=[
            pltpu.VMEM((128,), jnp.int32),
            pltpu.VMEM((LP - 128,), jnp.int32),
            pltpu.VMEM((LP, D), jnp.float32),
            pltpu.VMEM((OUT_CHUNK, D), jnp.float32),
            pltpu.SemaphoreType.DMA,
        ],
        compiler_params=pltpu.CompilerParams(use_tc_tiling_on_sc=False),
    )
    return f(xp, emb)


MLP_BLK = 1024


def _mlp_body(x_ref, sum_ref, e0_ref, w1_ref, b1_ref, w2_ref, b2_ref,
              w3_ref, b3_ref, out_ref):
    valid = jnp.sum((x_ref[...] != 0).astype(jnp.float32), axis=1,
                    keepdims=True)
    n0 = float(LP) - valid
    a = (sum_ref[...] - n0 * e0_ref[...]) / valid
    h = jnp.maximum(jnp.dot(a, w1_ref[...], preferred_element_type=jnp.float32)
                    + b1_ref[...], 0.0)
    h = jnp.maximum(jnp.dot(h, w2_ref[...], preferred_element_type=jnp.float32)
                    + b2_ref[...], 0.0)
    h = jnp.maximum(jnp.dot(h, w3_ref[...], preferred_element_type=jnp.float32)
                    + b3_ref[...], 0.0)
    m = jnp.max(h, axis=1, keepdims=True)
    lse = m + jnp.log(jnp.sum(jnp.exp(h - m), axis=1, keepdims=True))
    out_ref[...] = h - lse


@jax.jit
def _mlp(x, sums, e0, w1t, b1, w2t, b2, w3t, b3):
    full = lambda s: pl.BlockSpec(s, lambda i: (0,) * len(s))
    return pl.pallas_call(
        _mlp_body,
        grid=(B // MLP_BLK,),
        in_specs=[
            pl.BlockSpec((MLP_BLK, L), lambda i: (i, 0)),
            pl.BlockSpec((MLP_BLK, D), lambda i: (i, 0)),
            full(e0.shape),
            full(w1t.shape), full(b1.shape),
            full(w2t.shape), full(b2.shape),
            full(w3t.shape), full(b3.shape),
        ],
        out_specs=pl.BlockSpec((MLP_BLK, 2), lambda i: (i, 0)),
        out_shape=jax.ShapeDtypeStruct((B, 2), jnp.float32),
    )(x, sums, e0, w1t, b1, w2t, b2, w3t, b3)


def kernel(x, emb, W1, b1, W2, b2, W3, b3):
    xp = jnp.pad(x, ((0, 0), (0, LP - L)))
    sums = _sc_sum(xp, emb)
    return _mlp(x, sums, emb[0:1], W1.T, b1[None, :], W2.T, b2[None, :],
                W3.T, b3[None, :])


# speedup vs baseline: 3.7934x; 3.7689x over previous
import jax
import jax.numpy as jnp
from jax import lax
from jax.experimental import pallas as pl
from jax.experimental.pallas import tpu as pltpu
from jax.experimental.pallas import tpu_sc as plsc

B, L, D = 16384, 200, 64
LA, LB = 128, L - 128
NW = 32
RPW = B // NW
BLK = 128
NBLK = RPW // BLK
OUT_CHUNK = 16


NBUF = 4


def _sc_body(x_hbm, emb_hbm, sum_hbm, idx_h0, idx_h1, rows0, rows1, rows2,
             rows3, out_v, sem_i, sem_g0, sem_g1, sem_g2, sem_g3):
    ncores = lax.axis_size("c")
    wid = lax.axis_index("s") * ncores + lax.axis_index("c")
    base = wid * RPW
    rows = [rows0, rows1, rows2, rows3]
    sems = [sem_g0, sem_g1, sem_g2, sem_g3]

    def issue_row(ih, rr, rows_ref, sem):
        pltpu.async_copy(emb_hbm.at[ih.at[rr, pl.ds(0, LA)]],
                         rows_ref.at[pl.ds(0, LA)], sem)
        pltpu.async_copy(emb_hbm.at[ih.at[rr, pl.ds(LA, LB)]],
                         rows_ref.at[pl.ds(LA, LB)], sem)

    def wait_rows(rows_ref, sem):
        pltpu.make_async_copy(emb_hbm.at[pl.ds(0, L)], rows_ref, sem).wait()

    def acc_store(rows_ref, r):
        def acc8(j, accs):
            a0, a1, a2, a3 = accs
            for jj in range(8):
                row = j * 8 + jj
                a0 = a0 + rows_ref[row, pl.ds(0, 16)]
                a1 = a1 + rows_ref[row, pl.ds(16, 16)]
                a2 = a2 + rows_ref[row, pl.ds(32, 16)]
                a3 = a3 + rows_ref[row, pl.ds(48, 16)]
            return (a0, a1, a2, a3)

        z = jnp.zeros((16,), jnp.float32)
        a0, a1, a2, a3 = lax.fori_loop(0, L // 8, acc8, (z, z, z, z))
        s = r % OUT_CHUNK
        out_v[s, pl.ds(0, 16)] = a0
        out_v[s, pl.ds(16, 16)] = a1
        out_v[s, pl.ds(32, 16)] = a2
        out_v[s, pl.ds(48, 16)] = a3

        @pl.when(s == OUT_CHUNK - 1)
        def _flush():
            start = pl.multiple_of(base + r - (OUT_CHUNK - 1), OUT_CHUNK)
            pltpu.sync_copy(out_v, sum_hbm.at[pl.ds(start, OUT_CHUNK)])

    pltpu.sync_copy(x_hbm.at[pl.ds(pl.multiple_of(base, 8), BLK)], idx_h0)
    pltpu.async_copy(x_hbm.at[pl.ds(pl.multiple_of(base + BLK, 8), BLK)],
                     idx_h1, sem_i)
    for q in range(NBUF):
        issue_row(idx_h0, q, rows[q], sems[q])

    for h in range(NBLK):
        ih = idx_h0 if h % 2 == 0 else idx_h1

        def quad(i, _, ih=ih, h=h):
            for q in range(NBUF):
                rr = i * NBUF + q
                wait_rows(rows[q], sems[q])
                acc_store(rows[q], h * BLK + rr)

                @pl.when(i < BLK // NBUF - 1)
                def _refill(q=q, rr=rr):
                    issue_row(ih, rr + NBUF, rows[q], sems[q])

            return 0

        lax.fori_loop(0, BLK // NBUF, quad, 0)

        if h + 1 < NBLK:
            nih = idx_h1 if h % 2 == 0 else idx_h0
            pltpu.make_async_copy(x_hbm.at[pl.ds(0, BLK)], nih, sem_i).wait()
            for q in range(NBUF):
                issue_row(nih, q, rows[q], sems[q])
            if h + 2 < NBLK:
                pltpu.async_copy(
                    x_hbm.at[pl.ds(pl.multiple_of(base + (h + 2) * BLK, 8),
                                   BLK)], ih, sem_i)


@jax.jit
def _sc_sum(x, emb):
    mesh = plsc.VectorSubcoreMesh(core_axis_name="c", subcore_axis_name="s")
    f = pl.kernel(
        _sc_body,
        out_type=jax.ShapeDtypeStruct((B, D), jnp.float32),
        mesh=mesh,
        scratch_types=(
            [pltpu.VMEM((BLK, L), jnp.int32)] * 2
            + [pltpu.VMEM((L, D), jnp.float32)] * NBUF
            + [pltpu.VMEM((OUT_CHUNK, D), jnp.float32)]
            + [pltpu.SemaphoreType.DMA] * (1 + NBUF)
        ),
        compiler_params=pltpu.CompilerParams(use_tc_tiling_on_sc=False),
    )
    return f(x, emb)


MLP_BLK = 1024


def _mlp_body(x_ref, sum_ref, e0_ref, w1_ref, b1_ref, w2_ref, b2_ref,
              w3_ref, b3_ref, out_ref):
    valid = jnp.sum((x_ref[...] != 0).astype(jnp.float32), axis=1,
                    keepdims=True)
    n0 = float(L) - valid
    a = (sum_ref[...] - n0 * e0_ref[...]) / valid
    h = jnp.maximum(jnp.dot(a, w1_ref[...], preferred_element_type=jnp.float32)
                    + b1_ref[...], 0.0)
    h = jnp.maximum(jnp.dot(h, w2_ref[...], preferred_element_type=jnp.float32)
                    + b2_ref[...], 0.0)
    h = jnp.maximum(jnp.dot(h, w3_ref[...], preferred_element_type=jnp.float32)
                    + b3_ref[...], 0.0)
    m = jnp.max(h, axis=1, keepdims=True)
    lse = m + jnp.log(jnp.sum(jnp.exp(h - m), axis=1, keepdims=True))
    out_ref[...] = h - lse


@jax.jit
def _mlp(x, sums, e0, w1t, b1, w2t, b2, w3t, b3):
    full = lambda s: pl.BlockSpec(s, lambda i: (0,) * len(s))
    return pl.pallas_call(
        _mlp_body,
        grid=(B // MLP_BLK,),
        in_specs=[
            pl.BlockSpec((MLP_BLK, L), lambda i: (i, 0)),
            pl.BlockSpec((MLP_BLK, D), lambda i: (i, 0)),
            full(e0.shape),
            full(w1t.shape), full(b1.shape),
            full(w2t.shape), full(b2.shape),
            full(w3t.shape), full(b3.shape),
        ],
        out_specs=pl.BlockSpec((MLP_BLK, 2), lambda i: (i, 0)),
        out_shape=jax.ShapeDtypeStruct((B, 2), jnp.float32),
    )(x, sums, e0, w1t, b1, w2t, b2, w3t, b3)


def kernel(x, emb, W1, b1, W2, b2, W3, b3):
    sums = _sc_sum(x, emb)
    return _mlp(x, sums, emb[0:1], W1.T, b1[None, :], W2.T, b2[None, :],
                W3.T, b3[None, :])
